# baseline (speedup 1.0000x reference)
import functools

import jax
import jax.numpy as jnp
from jax.experimental import pallas as pl
from jax.experimental.pallas import tpu as pltpu

N = 8192
M = 8192
D_FEAT = 128
C = 64
TR = 256
TC = 256
NI = N // TR
NJ = M // TC


def _chamfer_band_kernel(sicl_s_ref, so_ref, eo_ref,
                         in_ref, incl_ref, out_ref, outcl_ref,
                         loss_ref, colmin_ref):
    nb = sicl_s_ref[N - 1]
    colmin_ref[...] = jnp.full((NJ, TC), jnp.inf, jnp.float32)

    def row_tile(t, acc):
        a = in_ref[pl.ds(t * TR, TR), :]
        a_sq = jnp.sum(a * a, axis=1)
        a16 = a.astype(jnp.bfloat16)
        icl = incl_ref[0, pl.ds(t * TR, TR)]

        c_lo = sicl_s_ref[t * TR]
        c_hi = sicl_s_ref[t * TR + TR - 1]
        jlo = so_ref[c_lo] // TC
        jhi = (eo_ref[c_hi] + TC - 1) // TC

        def body(j, rmin):
            b = out_ref[pl.ds(j * TC, TC), :]
            b_sq = jnp.sum(b * b, axis=1)
            ab = jax.lax.dot_general(
                a16, b.astype(jnp.bfloat16),
                (((1,), (1,)), ((), ())), preferred_element_type=jnp.float32)
            dist = a_sq[:, None] + b_sq[None, :] - 2.0 * ab
            ocl = outcl_ref[0, pl.ds(j * TC, TC)]
            dist = jnp.where(icl[:, None] == ocl[None, :], dist, jnp.inf)
            colmin_ref[j, :] = jnp.minimum(colmin_ref[j, :],
                                           jnp.min(dist, axis=0))
            return jnp.minimum(rmin, jnp.min(dist, axis=1))

        rmin0 = jnp.full((TR,), jnp.inf, jnp.float32)
        rmin = jax.lax.fori_loop(jlo, jhi, body, rmin0)
        return acc + jnp.sum(jnp.where(icl < nb, rmin, 0.0))

    loss = jax.lax.fori_loop(0, NI, row_tile, jnp.float32(0.0))

    def creduce(j, acc):
        ocl = outcl_ref[0, pl.ds(j * TC, TC)]
        return acc + jnp.sum(jnp.where(ocl < nb, colmin_ref[j, :], 0.0))

    loss_ref[0, 0] = loss + jax.lax.fori_loop(0, NJ, creduce,
                                              jnp.float32(0.0))


@jax.jit
def kernel(input_points, input_clusters, output_points, output_clusters):
    in_pts = input_points[0]
    out_pts = output_points[0]
    icl = input_clusters[0]
    ocl = output_clusters[0]

    sicl, order_in = jax.lax.sort([icl, jnp.arange(N, dtype=jnp.int32)],
                                  num_keys=1)
    socl, order_out = jax.lax.sort([ocl, jnp.arange(M, dtype=jnp.int32)],
                                   num_keys=1)
    sin = jnp.take(in_pts, order_in, axis=0)
    sout = jnp.take(out_pts, order_out, axis=0)

    cids = jnp.arange(C, dtype=jnp.int32)
    starts_out = jnp.searchsorted(socl, cids, side="left").astype(jnp.int32)
    ends_out = jnp.searchsorted(socl, cids, side="right").astype(jnp.int32)

    grid_spec = pltpu.PrefetchScalarGridSpec(
        num_scalar_prefetch=3,
        grid=(1,),
        in_specs=[
            pl.BlockSpec((N, D_FEAT), lambda i, *_: (0, 0)),
            pl.BlockSpec((1, N), lambda i, *_: (0, 0)),
            pl.BlockSpec((M, D_FEAT), lambda i, *_: (0, 0)),
            pl.BlockSpec((1, M), lambda i, *_: (0, 0)),
        ],
        out_specs=pl.BlockSpec(memory_space=pltpu.SMEM),
        scratch_shapes=[
            pltpu.VMEM((NJ, TC), jnp.float32),
        ],
    )
    loss = pl.pallas_call(
        _chamfer_band_kernel,
        grid_spec=grid_spec,
        out_shape=jax.ShapeDtypeStruct((1, 1), jnp.float32),
        compiler_params=pltpu.CompilerParams(
            dimension_semantics=("arbitrary",)),
    )(sicl, starts_out, ends_out,
      sin, sicl.reshape(1, N), sout, socl.reshape(1, M))
    return loss[0, 0]
